# baseline (speedup 1.0000x reference)
import jax
import jax.numpy as jnp
from jax.experimental import pallas as pl

_NUM_CLASSES = 21
_THRESHOLD = 0.5
_NEGPOS_RATIO = 3
_V0 = 0.1
_V1 = 0.2
_G = 4


def _match_ce_kernel(prior_ref, gt_ref, lab_ref, loc_ref, conf_ref, out_ref):
    PG = prior_ref.shape[2]
    NOBJ = gt_ref.shape[0]
    C = _NUM_CLASSES
    P = _G * PG
    pcx = prior_ref[0]
    pcy = prior_ref[1]
    pw = prior_ref[2]
    ph = prior_ref[3]
    gx1 = gt_ref[:, 0:1]
    gy1 = gt_ref[:, 1:2]
    gx2 = gt_ref[:, 2:3]
    gy2 = gt_ref[:, 3:4]
    labs = lab_ref[:, 0:1]
    area_g = (gx2 - gx1) * (gy2 - gy1)
    lt = loc_ref[...].T
    ct = conf_ref[...].T

    o_iota = jax.lax.broadcasted_iota(jnp.int32, (NOBJ, PG), 0)
    p_iota = jax.lax.broadcasted_iota(jnp.int32, (NOBJ, PG), 1)
    cls_iota = jax.lax.broadcasted_iota(jnp.int32, (C, PG), 0)

    ovs, m_os, bpis = [], [], []
    for a in range(_G):
        px1 = pcx[a:a + 1] - pw[a:a + 1] / 2.0
        py1 = pcy[a:a + 1] - ph[a:a + 1] / 2.0
        px2 = pcx[a:a + 1] + pw[a:a + 1] / 2.0
        py2 = pcy[a:a + 1] + ph[a:a + 1] / 2.0
        ix = jnp.clip(jnp.minimum(gx2, px2) - jnp.maximum(gx1, px1), 0.0, None)
        iy = jnp.clip(jnp.minimum(gy2, py2) - jnp.maximum(gy1, py1), 0.0, None)
        inter = ix * iy
        area_p = (px2 - px1) * (py2 - py1)
        ov = inter / (area_g + area_p - inter)
        ovs.append(ov)
        m_os.append(jnp.max(ov, axis=1, keepdims=True))
        aid = _G * p_iota + a
        bpis.append(jnp.min(jnp.where(ov == m_os[-1], aid, P), axis=1,
                            keepdims=True))
    m_o = jnp.maximum(jnp.maximum(m_os[0], m_os[1]),
                      jnp.maximum(m_os[2], m_os[3]))
    bpi = jnp.minimum(
        jnp.minimum(jnp.where(m_os[0] == m_o, bpis[0], P),
                    jnp.where(m_os[1] == m_o, bpis[1], P)),
        jnp.minimum(jnp.where(m_os[2] == m_o, bpis[2], P),
                    jnp.where(m_os[3] == m_o, bpis[3], P)))

    pieces = []
    for a in range(_G):
        ov = ovs[a]
        bto = jnp.max(ov, axis=0, keepdims=True)
        bti = jnp.min(jnp.where(ov == bto, o_iota, NOBJ), axis=0,
                      keepdims=True)
        aid = _G * p_iota + a
        eq = aid == bpi
        forced = jnp.max(jnp.where(eq, 1, 0), axis=0, keepdims=True) > 0
        fidx = jnp.max(jnp.where(eq, o_iota, -1), axis=0, keepdims=True)
        bti = jnp.where(forced, fidx, bti)
        bto = jnp.where(forced, 2.0, bto)
        sel = o_iota == bti
        mx1 = jnp.sum(jnp.where(sel, gx1, 0.0), axis=0, keepdims=True)
        my1 = jnp.sum(jnp.where(sel, gy1, 0.0), axis=0, keepdims=True)
        mx2 = jnp.sum(jnp.where(sel, gx2, 0.0), axis=0, keepdims=True)
        my2 = jnp.sum(jnp.where(sel, gy2, 0.0), axis=0, keepdims=True)
        mlab = jnp.sum(jnp.where(sel, labs, 0.0), axis=0, keepdims=True)
        conf_label = jnp.where(bto < _THRESHOLD, 0.0, mlab)
        pos = conf_label > 0.0
        pcxa, pcya = pcx[a:a + 1], pcy[a:a + 1]
        pwa, pha = pw[a:a + 1], ph[a:a + 1]
        gcx = ((mx1 + mx2) * 0.5 - pcxa) / (_V0 * pwa)
        gcy = ((my1 + my2) * 0.5 - pcya) / (_V0 * pha)
        gw = jnp.log(jnp.maximum(mx2 - mx1, 1e-6) / pwa) / _V1
        gh = jnp.log(jnp.maximum(my2 - my1, 1e-6) / pha) / _V1
        g = jnp.concatenate([gcx, gcy, gw, gh], axis=0)
        la = lt[4 * a:4 * (a + 1)]
        diff = la - g
        ad = jnp.abs(diff)
        sl1 = jnp.where(ad < 1.0, 0.5 * diff * diff, ad - 0.5)
        sl1_sum = jnp.sum(sl1, axis=0, keepdims=True)
        sl1_masked = jnp.where(pos, sl1_sum, 0.0)
        ca = ct[C * a:C * (a + 1)]
        cmax = jnp.max(ca, axis=0, keepdims=True)
        s = jnp.sum(jnp.exp(ca - cmax), axis=0, keepdims=True)
        lse = jnp.log(s) + cmax
        gathered = jnp.sum(
            jnp.where(cls_iota == conf_label.astype(jnp.int32), ca, 0.0),
            axis=0, keepdims=True)
        ce = lse - gathered
        ch = jnp.concatenate([jnp.where(pos, 0.0, ce),
                              jnp.where(pos, ce, 0.0),
                              sl1_masked,
                              pos.astype(jnp.float32)], axis=0)
        pieces.append(ch.reshape(4, 1, PG))
    out_ref[...] = jnp.concatenate(pieces, axis=1)


def _mine_reduce_kernel(ch_ref, out_ref):
    B = ch_ref.shape[0]
    P = ch_ref.shape[2]
    mine = ch_ref[:, 0, :]
    cepos = ch_ref[:, 1, :]
    sl1m = ch_ref[:, 2, :]
    posf = ch_ref[:, 3, :]
    num_pos = jnp.sum(posf, axis=1, keepdims=True)
    k = jnp.clip(_NEGPOS_RATIO * num_pos.astype(jnp.int32), 0, P - 1)
    u = jax.lax.bitcast_convert_type(mine, jnp.int32)
    inf_bits = jnp.int32(0x7F800000)

    def body(_, carry):
        lo, hi = carry
        mid = lo + (hi - lo + 1) // 2
        cnt = jnp.sum((u >= mid).astype(jnp.int32), axis=1, keepdims=True)
        ge = cnt >= k
        return jnp.where(ge, mid, lo), jnp.where(ge, hi, mid - 1)

    lo0 = jnp.zeros((B, 1), jnp.int32)
    hi0 = jnp.full((B, 1), inf_bits, jnp.int32)
    lo, _ = jax.lax.fori_loop(0, 32, body, (lo0, hi0))
    t = jax.lax.bitcast_convert_type(lo, jnp.float32)
    gt = mine > t
    cnt_gt = jnp.sum(jnp.where(gt, 1.0, 0.0), axis=1, keepdims=True)
    sum_gt = jnp.sum(jnp.where(gt, mine, 0.0), axis=1, keepdims=True)
    topk = sum_gt + t * (k.astype(jnp.float32) - cnt_gt)
    topk = jnp.where(k > 0, topk, 0.0)
    loss_c_rows = jnp.sum(cepos, axis=1, keepdims=True) + topk
    loss_l_rows = jnp.sum(sl1m, axis=1, keepdims=True)
    n = jnp.maximum(jnp.sum(num_pos, axis=0, keepdims=True), 1.0)
    ll = jnp.sum(loss_l_rows, axis=0, keepdims=True) / n
    lc = jnp.sum(loss_c_rows, axis=0, keepdims=True) / n
    out_ref[...] = jnp.concatenate([ll, lc], axis=1)


@jax.jit
def kernel(loc_data, conf_data, priors, gt_boxes, gt_labels):
    B, P, C = conf_data.shape
    NOBJ = gt_boxes.shape[1]
    PG = P // _G
    conf_v = conf_data.reshape(B, PG, _G * C)
    loc_v = loc_data.reshape(B, PG, _G * 4)
    priors_v = priors.T.reshape(4, PG, _G).transpose(0, 2, 1)
    labels_f = gt_labels.astype(jnp.float32).reshape(B, NOBJ, 1)

    channels = pl.pallas_call(
        _match_ce_kernel,
        grid=(B,),
        in_specs=[
            pl.BlockSpec((4, _G, PG), lambda b: (0, 0, 0)),
            pl.BlockSpec((None, NOBJ, 4), lambda b: (b, 0, 0)),
            pl.BlockSpec((None, NOBJ, 1), lambda b: (b, 0, 0)),
            pl.BlockSpec((None, PG, _G * 4), lambda b: (b, 0, 0)),
            pl.BlockSpec((None, PG, _G * C), lambda b: (b, 0, 0)),
        ],
        out_specs=pl.BlockSpec((None, 4, _G, PG), lambda b: (b, 0, 0, 0)),
        out_shape=jax.ShapeDtypeStruct((B, 4, _G, PG), jnp.float32),
    )(priors_v, gt_boxes, labels_f, loc_v, conf_v)

    ch = channels.reshape(B, 4, P)
    out = pl.pallas_call(
        _mine_reduce_kernel,
        in_specs=[pl.BlockSpec((B, 4, P), lambda: (0, 0, 0))],
        out_specs=pl.BlockSpec((1, 2), lambda: (0, 0)),
        out_shape=jax.ShapeDtypeStruct((1, 2), jnp.float32),
    )(ch)
    return out.reshape(2)


# speedup vs baseline: 2.3633x; 2.3633x over previous
import jax
import jax.numpy as jnp
from jax.experimental import pallas as pl
from jax.experimental.pallas import tpu as pltpu

_NUM_CLASSES = 21
_THRESHOLD = 0.5
_NEGPOS_RATIO = 3
_V0 = 0.1
_V1 = 0.2


def _match_ce_kernel(prior_ref, gt_ref, lab_ref, loc_ref, conf_ref, out_ref):
    P = prior_ref.shape[1]
    NOBJ = gt_ref.shape[0]
    pcx = prior_ref[0:1, :]
    pcy = prior_ref[1:2, :]
    pw = prior_ref[2:3, :]
    ph = prior_ref[3:4, :]
    px1 = pcx - pw / 2.0
    py1 = pcy - ph / 2.0
    px2 = pcx + pw / 2.0
    py2 = pcy + ph / 2.0
    gx1 = gt_ref[:, 0:1]
    gy1 = gt_ref[:, 1:2]
    gx2 = gt_ref[:, 2:3]
    gy2 = gt_ref[:, 3:4]
    labs = lab_ref[:, 0:1]
    ix = jnp.clip(jnp.minimum(gx2, px2) - jnp.maximum(gx1, px1), 0.0, None)
    iy = jnp.clip(jnp.minimum(gy2, py2) - jnp.maximum(gy1, py1), 0.0, None)
    inter = ix * iy
    area_g = (gx2 - gx1) * (gy2 - gy1)
    area_p = (px2 - px1) * (py2 - py1)
    ov = inter / (area_g + area_p - inter)

    o_iota = jax.lax.broadcasted_iota(jnp.int32, (NOBJ, P), 0)
    p_iota = jax.lax.broadcasted_iota(jnp.int32, (NOBJ, P), 1)

    bto = jnp.max(ov, axis=0, keepdims=True)
    bti = jnp.min(jnp.where(ov == bto, o_iota, NOBJ), axis=0, keepdims=True)
    m_o = jnp.max(ov, axis=1, keepdims=True)
    bpi = jnp.min(jnp.where(ov == m_o, p_iota, P), axis=1, keepdims=True)
    eq = p_iota == bpi
    forced = jnp.max(jnp.where(eq, 1, 0), axis=0, keepdims=True) > 0
    fidx = jnp.max(jnp.where(eq, o_iota, -1), axis=0, keepdims=True)
    bti = jnp.where(forced, fidx, bti)
    bto = jnp.where(forced, 2.0, bto)
    sel = o_iota == bti
    mx1 = jnp.sum(jnp.where(sel, gx1, 0.0), axis=0, keepdims=True)
    my1 = jnp.sum(jnp.where(sel, gy1, 0.0), axis=0, keepdims=True)
    mx2 = jnp.sum(jnp.where(sel, gx2, 0.0), axis=0, keepdims=True)
    my2 = jnp.sum(jnp.where(sel, gy2, 0.0), axis=0, keepdims=True)
    mlab = jnp.sum(jnp.where(sel, labs, 0.0), axis=0, keepdims=True)
    conf_label = jnp.where(bto < _THRESHOLD, 0.0, mlab)
    pos = conf_label > 0.0
    gcx = ((mx1 + mx2) * 0.5 - pcx) / (_V0 * pw)
    gcy = ((my1 + my2) * 0.5 - pcy) / (_V0 * ph)
    gw = jnp.log(jnp.maximum(mx2 - mx1, 1e-6) / pw) / _V1
    gh = jnp.log(jnp.maximum(my2 - my1, 1e-6) / ph) / _V1
    g = jnp.concatenate([gcx, gcy, gw, gh], axis=0)
    diff = loc_ref[...] - g
    ad = jnp.abs(diff)
    sl1 = jnp.where(ad < 1.0, 0.5 * diff * diff, ad - 0.5)
    sl1_sum = jnp.sum(sl1, axis=0, keepdims=True)
    sl1_masked = jnp.where(pos, sl1_sum, 0.0)
    c = conf_ref[...]
    cmax = jnp.max(c, axis=0, keepdims=True)
    s = jnp.sum(jnp.exp(c - cmax), axis=0, keepdims=True)
    lse = jnp.log(s) + cmax
    cls_iota = jax.lax.broadcasted_iota(jnp.int32, (c.shape[0], P), 0)
    gathered = jnp.sum(jnp.where(cls_iota == conf_label.astype(jnp.int32),
                                 c, 0.0), axis=0, keepdims=True)
    ce = lse - gathered
    out_ref[0:1, :] = jnp.where(pos, 0.0, ce)
    out_ref[1:2, :] = jnp.where(pos, ce, 0.0)
    out_ref[2:3, :] = sl1_masked
    out_ref[3:4, :] = pos.astype(jnp.float32)


def _mine_reduce_kernel(ch_ref, out_ref):
    B = ch_ref.shape[0]
    P = ch_ref.shape[2]
    mine = ch_ref[:, 0, :]
    cepos = ch_ref[:, 1, :]
    sl1m = ch_ref[:, 2, :]
    posf = ch_ref[:, 3, :]
    num_pos = jnp.sum(posf, axis=1, keepdims=True)
    k = jnp.clip(_NEGPOS_RATIO * num_pos.astype(jnp.int32), 0, P - 1)
    u = jax.lax.bitcast_convert_type(mine, jnp.int32)
    inf_bits = jnp.int32(0x7F800000)

    def body(_, carry):
        lo, hi = carry
        mid = lo + (hi - lo + 1) // 2
        cnt = jnp.sum((u >= mid).astype(jnp.int32), axis=1, keepdims=True)
        ge = cnt >= k
        return jnp.where(ge, mid, lo), jnp.where(ge, hi, mid - 1)

    lo0 = jnp.zeros((B, 1), jnp.int32)
    hi0 = jnp.full((B, 1), inf_bits, jnp.int32)
    lo, _ = jax.lax.fori_loop(0, 32, body, (lo0, hi0))
    t = jax.lax.bitcast_convert_type(lo, jnp.float32)
    gt = mine > t
    cnt_gt = jnp.sum(jnp.where(gt, 1.0, 0.0), axis=1, keepdims=True)
    sum_gt = jnp.sum(jnp.where(gt, mine, 0.0), axis=1, keepdims=True)
    topk = sum_gt + t * (k.astype(jnp.float32) - cnt_gt)
    topk = jnp.where(k > 0, topk, 0.0)
    loss_c_rows = jnp.sum(cepos, axis=1, keepdims=True) + topk
    loss_l_rows = jnp.sum(sl1m, axis=1, keepdims=True)
    n = jnp.maximum(jnp.sum(num_pos, axis=0, keepdims=True), 1.0)
    ll = jnp.sum(loss_l_rows, axis=0, keepdims=True) / n
    lc = jnp.sum(loss_c_rows, axis=0, keepdims=True) / n
    out_ref[...] = jnp.concatenate([ll, lc], axis=1)


@jax.jit
def kernel(loc_data, conf_data, priors, gt_boxes, gt_labels):
    B, P, C = conf_data.shape
    NOBJ = gt_boxes.shape[1]
    conf_t = jnp.transpose(conf_data, (0, 2, 1))
    loc_t = jnp.transpose(loc_data, (0, 2, 1))
    priors_t = priors.T
    labels_f = gt_labels.astype(jnp.float32).reshape(B, NOBJ, 1)

    channels = pl.pallas_call(
        _match_ce_kernel,
        grid=(B,),
        in_specs=[
            pl.BlockSpec((4, P), lambda b: (0, 0)),
            pl.BlockSpec((None, NOBJ, 4), lambda b: (b, 0, 0)),
            pl.BlockSpec((None, NOBJ, 1), lambda b: (b, 0, 0)),
            pl.BlockSpec((None, 4, P), lambda b: (b, 0, 0)),
            pl.BlockSpec((None, C, P), lambda b: (b, 0, 0)),
        ],
        out_specs=pl.BlockSpec((None, 4, P), lambda b: (b, 0, 0)),
        out_shape=jax.ShapeDtypeStruct((B, 4, P), jnp.float32),
        compiler_params=pltpu.CompilerParams(
            dimension_semantics=("parallel",)),
    )(priors_t, gt_boxes, labels_f, loc_t, conf_t)

    out = pl.pallas_call(
        _mine_reduce_kernel,
        in_specs=[pl.BlockSpec((B, 4, P), lambda: (0, 0, 0))],
        out_specs=pl.BlockSpec((1, 2), lambda: (0, 0)),
        out_shape=jax.ShapeDtypeStruct((1, 2), jnp.float32),
    )(channels)
    return out.reshape(2)


# speedup vs baseline: 2.8212x; 1.1938x over previous
import jax
import jax.numpy as jnp
from jax.experimental import pallas as pl
from jax.experimental.pallas import tpu as pltpu

_NUM_CLASSES = 21
_THRESHOLD = 0.5
_NEGPOS_RATIO = 3
_V0 = 0.1
_V1 = 0.2


def _match_ce_kernel(prior_ref, gt_ref, lab_ref, loc_ref, conf_ref, out_ref):
    P = prior_ref.shape[1]
    NOBJ = gt_ref.shape[0]
    pcx = prior_ref[0:1, :]
    pcy = prior_ref[1:2, :]
    pw = prior_ref[2:3, :]
    ph = prior_ref[3:4, :]
    px1 = pcx - pw / 2.0
    py1 = pcy - ph / 2.0
    px2 = pcx + pw / 2.0
    py2 = pcy + ph / 2.0
    gx1 = gt_ref[:, 0:1]
    gy1 = gt_ref[:, 1:2]
    gx2 = gt_ref[:, 2:3]
    gy2 = gt_ref[:, 3:4]
    labs = lab_ref[:, 0:1]
    ix = jnp.clip(jnp.minimum(gx2, px2) - jnp.maximum(gx1, px1), 0.0, None)
    iy = jnp.clip(jnp.minimum(gy2, py2) - jnp.maximum(gy1, py1), 0.0, None)
    inter = ix * iy
    area_g = (gx2 - gx1) * (gy2 - gy1)
    area_p = (px2 - px1) * (py2 - py1)
    ov = inter / (area_g + area_p - inter)

    o_iota = jax.lax.broadcasted_iota(jnp.int32, (NOBJ, P), 0)
    p_iota = jax.lax.broadcasted_iota(jnp.int32, (NOBJ, P), 1)

    bto = jnp.max(ov, axis=0, keepdims=True)
    bti = jnp.min(jnp.where(ov == bto, o_iota, NOBJ), axis=0, keepdims=True)
    m_o = jnp.max(ov, axis=1, keepdims=True)
    bpi = jnp.min(jnp.where(ov == m_o, p_iota, P), axis=1, keepdims=True)
    eq = p_iota == bpi
    forced = jnp.max(jnp.where(eq, 1, 0), axis=0, keepdims=True) > 0
    fidx = jnp.max(jnp.where(eq, o_iota, -1), axis=0, keepdims=True)
    bti = jnp.where(forced, fidx, bti)
    bto = jnp.where(forced, 2.0, bto)
    sel_f = (o_iota == bti).astype(jnp.float32)
    gt_tab = jnp.concatenate([gx1, gy1, gx2, gy2, labs], axis=1).T
    m5 = jax.lax.dot_general(gt_tab, sel_f, (((1,), (0,)), ((), ())),
                             preferred_element_type=jnp.float32)
    mx1 = m5[0:1, :]
    my1 = m5[1:2, :]
    mx2 = m5[2:3, :]
    my2 = m5[3:4, :]
    mlab = m5[4:5, :]
    conf_label = jnp.where(bto < _THRESHOLD, 0.0, mlab)
    pos = conf_label > 0.0
    gcx = ((mx1 + mx2) * 0.5 - pcx) / (_V0 * pw)
    gcy = ((my1 + my2) * 0.5 - pcy) / (_V0 * ph)
    gw = jnp.log(jnp.maximum(mx2 - mx1, 1e-6) / pw) / _V1
    gh = jnp.log(jnp.maximum(my2 - my1, 1e-6) / ph) / _V1
    g = jnp.concatenate([gcx, gcy, gw, gh], axis=0)
    diff = loc_ref[...] - g
    ad = jnp.abs(diff)
    sl1 = jnp.where(ad < 1.0, 0.5 * diff * diff, ad - 0.5)
    sl1_sum = jnp.sum(sl1, axis=0, keepdims=True)
    sl1_masked = jnp.where(pos, sl1_sum, 0.0)
    c = conf_ref[...]
    cmax = jnp.max(c, axis=0, keepdims=True)
    e = jnp.exp(c - cmax)
    ones_c = jnp.ones((1, c.shape[0]), jnp.float32)
    s = jax.lax.dot_general(ones_c, e, (((1,), (0,)), ((), ())),
                            preferred_element_type=jnp.float32)
    lse = jnp.log(s) + cmax
    cls_iota = jax.lax.broadcasted_iota(jnp.int32, (c.shape[0], P), 0)
    gathered = jnp.sum(jnp.where(cls_iota == conf_label.astype(jnp.int32),
                                 c, 0.0), axis=0, keepdims=True)
    ce = lse - gathered
    out_ref[0:1, :] = jnp.where(pos, 0.0, ce)
    out_ref[1:2, :] = jnp.where(pos, ce, 0.0)
    out_ref[2:3, :] = sl1_masked
    out_ref[3:4, :] = pos.astype(jnp.float32)


def _mine_reduce_kernel(ch_ref, out_ref):
    B = ch_ref.shape[0]
    P = ch_ref.shape[2]
    mine = ch_ref[:, 0, :]
    cepos = ch_ref[:, 1, :]
    sl1m = ch_ref[:, 2, :]
    posf = ch_ref[:, 3, :]
    num_pos = jnp.sum(posf, axis=1, keepdims=True)
    k = jnp.clip(_NEGPOS_RATIO * num_pos.astype(jnp.int32), 0, P - 1)
    u = jax.lax.bitcast_convert_type(mine, jnp.int32)
    inf_bits = jnp.int32(0x7F800000)

    def body(_, carry):
        lo, hi = carry
        mid = lo + (hi - lo + 1) // 2
        cnt = jnp.sum((u >= mid).astype(jnp.int32), axis=1, keepdims=True)
        ge = cnt >= k
        return jnp.where(ge, mid, lo), jnp.where(ge, hi, mid - 1)

    lo0 = jnp.zeros((B, 1), jnp.int32)
    hi0 = jnp.full((B, 1), inf_bits, jnp.int32)
    lo, _ = jax.lax.fori_loop(0, 32, body, (lo0, hi0))
    t = jax.lax.bitcast_convert_type(lo, jnp.float32)
    gt = mine > t
    cnt_gt = jnp.sum(jnp.where(gt, 1.0, 0.0), axis=1, keepdims=True)
    sum_gt = jnp.sum(jnp.where(gt, mine, 0.0), axis=1, keepdims=True)
    topk = sum_gt + t * (k.astype(jnp.float32) - cnt_gt)
    topk = jnp.where(k > 0, topk, 0.0)
    loss_c_rows = jnp.sum(cepos, axis=1, keepdims=True) + topk
    loss_l_rows = jnp.sum(sl1m, axis=1, keepdims=True)
    n = jnp.maximum(jnp.sum(num_pos, axis=0, keepdims=True), 1.0)
    ll = jnp.sum(loss_l_rows, axis=0, keepdims=True) / n
    lc = jnp.sum(loss_c_rows, axis=0, keepdims=True) / n
    out_ref[...] = jnp.concatenate([ll, lc], axis=1)


@jax.jit
def kernel(loc_data, conf_data, priors, gt_boxes, gt_labels):
    B, P, C = conf_data.shape
    NOBJ = gt_boxes.shape[1]
    conf_t = jnp.transpose(conf_data, (0, 2, 1))
    loc_t = jnp.transpose(loc_data, (0, 2, 1))
    priors_t = priors.T
    labels_f = gt_labels.astype(jnp.float32).reshape(B, NOBJ, 1)

    channels = pl.pallas_call(
        _match_ce_kernel,
        grid=(B,),
        in_specs=[
            pl.BlockSpec((4, P), lambda b: (0, 0)),
            pl.BlockSpec((None, NOBJ, 4), lambda b: (b, 0, 0)),
            pl.BlockSpec((None, NOBJ, 1), lambda b: (b, 0, 0)),
            pl.BlockSpec((None, 4, P), lambda b: (b, 0, 0)),
            pl.BlockSpec((None, C, P), lambda b: (b, 0, 0)),
        ],
        out_specs=pl.BlockSpec((None, 4, P), lambda b: (b, 0, 0)),
        out_shape=jax.ShapeDtypeStruct((B, 4, P), jnp.float32),
        compiler_params=pltpu.CompilerParams(
            dimension_semantics=("parallel",)),
    )(priors_t, gt_boxes, labels_f, loc_t, conf_t)

    out = pl.pallas_call(
        _mine_reduce_kernel,
        in_specs=[pl.BlockSpec((B, 4, P), lambda: (0, 0, 0))],
        out_specs=pl.BlockSpec((1, 2), lambda: (0, 0)),
        out_shape=jax.ShapeDtypeStruct((1, 2), jnp.float32),
    )(channels)
    return out.reshape(2)


# speedup vs baseline: 3.0256x; 1.0724x over previous
import jax
import jax.numpy as jnp
from jax.experimental import pallas as pl

_NUM_CLASSES = 21
_THRESHOLD = 0.5
_NEGPOS_RATIO = 3
_V0 = 0.1
_V1 = 0.2


def _match_kernel(prior_ref, gt_ref, lab_ref, loc_ref, out_ref):
    P = prior_ref.shape[1]
    NOBJ = gt_ref.shape[0]
    pcx = prior_ref[0:1, :]
    pcy = prior_ref[1:2, :]
    pw = prior_ref[2:3, :]
    ph = prior_ref[3:4, :]
    px1 = pcx - pw / 2.0
    py1 = pcy - ph / 2.0
    px2 = pcx + pw / 2.0
    py2 = pcy + ph / 2.0
    gx1 = gt_ref[:, 0:1]
    gy1 = gt_ref[:, 1:2]
    gx2 = gt_ref[:, 2:3]
    gy2 = gt_ref[:, 3:4]
    labs = lab_ref[:, 0:1]
    ix = jnp.clip(jnp.minimum(gx2, px2) - jnp.maximum(gx1, px1), 0.0, None)
    iy = jnp.clip(jnp.minimum(gy2, py2) - jnp.maximum(gy1, py1), 0.0, None)
    inter = ix * iy
    area_g = (gx2 - gx1) * (gy2 - gy1)
    area_p = (px2 - px1) * (py2 - py1)
    ov = inter / (area_g + area_p - inter)

    o_iota = jax.lax.broadcasted_iota(jnp.int32, (NOBJ, P), 0)
    p_iota = jax.lax.broadcasted_iota(jnp.int32, (NOBJ, P), 1)

    bto = jnp.max(ov, axis=0, keepdims=True)
    bti = jnp.min(jnp.where(ov == bto, o_iota, NOBJ), axis=0, keepdims=True)
    m_o = jnp.max(ov, axis=1, keepdims=True)
    bpi = jnp.min(jnp.where(ov == m_o, p_iota, P), axis=1, keepdims=True)
    eq = p_iota == bpi
    forced = jnp.max(jnp.where(eq, 1, 0), axis=0, keepdims=True) > 0
    fidx = jnp.max(jnp.where(eq, o_iota, -1), axis=0, keepdims=True)
    bti = jnp.where(forced, fidx, bti)
    bto = jnp.where(forced, 2.0, bto)
    sel_f = (o_iota == bti).astype(jnp.float32)
    gt_tab = jnp.concatenate([gx1, gy1, gx2, gy2, labs], axis=1).T
    m5 = jax.lax.dot_general(gt_tab, sel_f, (((1,), (0,)), ((), ())),
                             preferred_element_type=jnp.float32)
    mx1 = m5[0:1, :]
    my1 = m5[1:2, :]
    mx2 = m5[2:3, :]
    my2 = m5[3:4, :]
    mlab = m5[4:5, :]
    conf_label = jnp.where(bto < _THRESHOLD, 0.0, mlab)
    pos = conf_label > 0.0
    gcx = ((mx1 + mx2) * 0.5 - pcx) / (_V0 * pw)
    gcy = ((my1 + my2) * 0.5 - pcy) / (_V0 * ph)
    gw = jnp.log(jnp.maximum(mx2 - mx1, 1e-6) / pw) / _V1
    gh = jnp.log(jnp.maximum(my2 - my1, 1e-6) / ph) / _V1
    g = jnp.concatenate([gcx, gcy, gw, gh], axis=0)
    diff = loc_ref[...] - g
    ad = jnp.abs(diff)
    sl1 = jnp.where(ad < 1.0, 0.5 * diff * diff, ad - 0.5)
    sl1_sum = jnp.sum(sl1, axis=0, keepdims=True)
    out_ref[0:1, :] = conf_label
    out_ref[1:2, :] = jnp.where(pos, sl1_sum, 0.0)


def _ce_kernel(conf_ref, m_ref, out_ref):
    P = conf_ref.shape[1]
    conf_label = m_ref[0:1, :]
    pos = conf_label > 0.0
    c = conf_ref[...]
    cmax = jnp.max(c, axis=0, keepdims=True)
    e = jnp.exp(c - cmax)
    ones_c = jnp.ones((1, c.shape[0]), jnp.float32)
    s = jax.lax.dot_general(ones_c, e, (((1,), (0,)), ((), ())),
                            preferred_element_type=jnp.float32)
    lse = jnp.log(s) + cmax
    cls_iota = jax.lax.broadcasted_iota(jnp.int32, (c.shape[0], P), 0)
    gathered = jnp.sum(jnp.where(cls_iota == conf_label.astype(jnp.int32),
                                 c, 0.0), axis=0, keepdims=True)
    ce = lse - gathered
    out_ref[0:1, :] = jnp.where(pos, 0.0, ce)
    out_ref[1:2, :] = jnp.where(pos, ce, 0.0)
    out_ref[2:3, :] = m_ref[1:2, :]
    out_ref[3:4, :] = pos.astype(jnp.float32)


def _mine_reduce_kernel(ch_ref, out_ref):
    B = ch_ref.shape[0]
    P = ch_ref.shape[2]
    mine = ch_ref[:, 0, :]
    cepos = ch_ref[:, 1, :]
    sl1m = ch_ref[:, 2, :]
    posf = ch_ref[:, 3, :]
    num_pos = jnp.sum(posf, axis=1, keepdims=True)
    k = jnp.clip(_NEGPOS_RATIO * num_pos.astype(jnp.int32), 0, P - 1)
    u = jax.lax.bitcast_convert_type(mine, jnp.int32)
    inf_bits = jnp.int32(0x7F800000)

    def body(_, carry):
        lo, hi = carry
        mid = lo + (hi - lo + 1) // 2
        cnt = jnp.sum((u >= mid).astype(jnp.int32), axis=1, keepdims=True)
        ge = cnt >= k
        return jnp.where(ge, mid, lo), jnp.where(ge, hi, mid - 1)

    lo0 = jnp.zeros((B, 1), jnp.int32)
    hi0 = jnp.full((B, 1), inf_bits, jnp.int32)
    lo, _ = jax.lax.fori_loop(0, 32, body, (lo0, hi0))
    t = jax.lax.bitcast_convert_type(lo, jnp.float32)
    gt = mine > t
    cnt_gt = jnp.sum(jnp.where(gt, 1.0, 0.0), axis=1, keepdims=True)
    sum_gt = jnp.sum(jnp.where(gt, mine, 0.0), axis=1, keepdims=True)
    topk = sum_gt + t * (k.astype(jnp.float32) - cnt_gt)
    topk = jnp.where(k > 0, topk, 0.0)
    loss_c_rows = jnp.sum(cepos, axis=1, keepdims=True) + topk
    loss_l_rows = jnp.sum(sl1m, axis=1, keepdims=True)
    n = jnp.maximum(jnp.sum(num_pos, axis=0, keepdims=True), 1.0)
    ll = jnp.sum(loss_l_rows, axis=0, keepdims=True) / n
    lc = jnp.sum(loss_c_rows, axis=0, keepdims=True) / n
    out_ref[...] = jnp.concatenate([ll, lc], axis=1)


@jax.jit
def kernel(loc_data, conf_data, priors, gt_boxes, gt_labels):
    B, P, C = conf_data.shape
    NOBJ = gt_boxes.shape[1]
    conf_t = jnp.transpose(conf_data, (0, 2, 1))
    loc_t = jnp.transpose(loc_data, (0, 2, 1))
    priors_t = priors.T
    labels_f = gt_labels.astype(jnp.float32).reshape(B, NOBJ, 1)

    match_ch = pl.pallas_call(
        _match_kernel,
        grid=(B,),
        in_specs=[
            pl.BlockSpec((4, P), lambda b: (0, 0)),
            pl.BlockSpec((None, NOBJ, 4), lambda b: (b, 0, 0)),
            pl.BlockSpec((None, NOBJ, 1), lambda b: (b, 0, 0)),
            pl.BlockSpec((None, 4, P), lambda b: (b, 0, 0)),
        ],
        out_specs=pl.BlockSpec((None, 2, P), lambda b: (b, 0, 0)),
        out_shape=jax.ShapeDtypeStruct((B, 2, P), jnp.float32),
    )(priors_t, gt_boxes, labels_f, loc_t)

    channels = pl.pallas_call(
        _ce_kernel,
        grid=(B,),
        in_specs=[
            pl.BlockSpec((None, C, P), lambda b: (b, 0, 0)),
            pl.BlockSpec((None, 2, P), lambda b: (b, 0, 0)),
        ],
        out_specs=pl.BlockSpec((None, 4, P), lambda b: (b, 0, 0)),
        out_shape=jax.ShapeDtypeStruct((B, 4, P), jnp.float32),
    )(conf_t, match_ch)

    out = pl.pallas_call(
        _mine_reduce_kernel,
        in_specs=[pl.BlockSpec((B, 4, P), lambda: (0, 0, 0))],
        out_specs=pl.BlockSpec((1, 2), lambda: (0, 0)),
        out_shape=jax.ShapeDtypeStruct((1, 2), jnp.float32),
    )(channels)
    return out.reshape(2)
